# baseline (speedup 1.0000x reference)
import jax
import jax.numpy as jnp
from jax import lax
from jax.experimental import pallas as pl
from jax.experimental.pallas import tpu as pltpu

F32 = jnp.float32
B, C, T = 16, 256, 256
HIGH = lax.Precision.HIGHEST


def _conv_mm(h, w_ref, b_ref):
    hm1 = jnp.concatenate([jnp.zeros((C, 1), F32), h[:, :-1]], axis=1)
    hp1 = jnp.concatenate([h[:, 1:], jnp.zeros((C, 1), F32)], axis=1)
    acc = jnp.dot(w_ref[0:C, :], hm1, preferred_element_type=F32, precision=HIGH)
    acc = acc + jnp.dot(w_ref[C:2 * C, :], h, preferred_element_type=F32, precision=HIGH)
    acc = acc + jnp.dot(w_ref[2 * C:3 * C, :], hp1, preferred_element_type=F32, precision=HIGH)
    return acc + b_ref[:]


def _layer_norm(h, g_ref, be_ref):
    mu = jnp.mean(h, axis=1, keepdims=True)
    hc = h - mu
    var = jnp.mean(hc * hc, axis=1, keepdims=True)
    return hc * lax.rsqrt(var + 1e-5) * g_ref[:] + be_ref[:]


def _vp(h0, w1, b1, g1, be1, w2, b2, g2, be2, lw, lb):
    h = jnp.maximum(_conv_mm(h0, w1, b1), 0.0)
    h = _layer_norm(h, g1, be1)
    h = jnp.maximum(_conv_mm(h, w2, b2), 0.0)
    h = _layer_norm(h, g2, be2)
    pred = lax.dot_general(lw[:], h, (((1,), (1,)), ((), ())),
                           preferred_element_type=F32, precision=HIGH)
    return pred + lb[:]


def _emb(t_col, w_ref, b_ref):
    tm1 = jnp.concatenate([jnp.zeros((1, 1), F32), t_col[:-1, :]], axis=0)
    tp1 = jnp.concatenate([t_col[1:, :], jnp.zeros((1, 1), F32)], axis=0)
    return (tm1 * w_ref[0:1, :] + t_col * w_ref[1:2, :] + tp1 * w_ref[2:3, :]
            + b_ref[:])


def _body(maxlen_ref, x_ref, dur_ref, pt_ref, et_ref,
          dw1, db1, dg1, dbe1, dw2, db2, dg2, dbe2, dlw, dlb,
          pw1, pb1, pg1, pbe1, pw2, pb2, pg2, pbe2, plw, plb,
          ew1, eb1, eg1, ebe1, ew2, eb2, eg2, ebe2, elw, elb,
          pew, peb, eew, eeb,
          out_ref, mel_ref, dpred_ref, ppred_ref, epred_ref):
    xb = x_ref[0]

    dpred_ref[0] = _vp(xb, dw1, db1, dg1, dbe1, dw2, db2, dg2, dbe2, dlw, dlb)

    d = dur_ref[0].astype(F32)
    row_i = lax.broadcasted_iota(jnp.int32, (C, C), 0).astype(F32)
    col_i = lax.broadcasted_iota(jnp.int32, (C, C), 1).astype(F32)
    upper = (row_i <= col_i).astype(F32)
    cs = jnp.dot(d, upper, preferred_element_type=F32, precision=HIGH)
    total = jnp.sum(d)
    idx = jnp.sum((row_i >= cs).astype(F32), axis=1, keepdims=True)
    pcol = lax.broadcasted_iota(jnp.int32, (C, 1), 0).astype(F32)
    maxlen_f = maxlen_ref[0].astype(F32)
    valid = (pcol < total) & (pcol < maxlen_f)
    onehot = ((idx == col_i) & valid).astype(F32)
    x2b = jnp.dot(onehot, xb, preferred_element_type=F32, precision=HIGH)
    mel_ref[0] = jnp.full((1, C), total, F32).astype(jnp.int32)

    ppred_ref[0] = _vp(x2b, pw1, pb1, pg1, pbe1, pw2, pb2, pg2, pbe2, plw, plb)
    epred_ref[0] = _vp(x2b, ew1, eb1, eg1, ebe1, ew2, eb2, eg2, ebe2, elw, elb)

    pemb = _emb(pt_ref[0], pew, peb)
    eemb = _emb(et_ref[0], eew, eeb)
    out_ref[0] = x2b + pemb + eemb


def _stack_conv_w(w):
    return jnp.concatenate([w[:, :, 0], w[:, :, 1], w[:, :, 2]], axis=0)


def _full(shape):
    nd = len(shape)
    return pl.BlockSpec(shape, lambda b: (0,) * nd)


def kernel(x, src_len, duration_target, pitch_target, energy_target, max_len,
           dp_w1, dp_b1, dp_g1, dp_be1, dp_w2, dp_b2, dp_g2, dp_be2, dp_lw, dp_lb,
           pp_w1, pp_b1, pp_g1, pp_be1, pp_w2, pp_b2, pp_g2, pp_be2, pp_lw, pp_lb,
           ep_w1, ep_b1, ep_g1, ep_be1, ep_w2, ep_b2, ep_g2, ep_be2, ep_lw, ep_lb,
           pe_w, pe_b, ee_w, ee_b):
    del src_len
    dur3 = duration_target.astype(jnp.int32).reshape(B, 1, T)
    pt3 = pitch_target.astype(F32).reshape(B, T, 1)
    et3 = energy_target.astype(F32).reshape(B, T, 1)
    maxlen = jnp.asarray(max_len, jnp.int32).reshape(1)

    def prep_vp(w1, b1, g1, be1, w2, b2, g2, be2, lw, lb):
        return (_stack_conv_w(w1), b1.reshape(C, 1), g1.reshape(1, C),
                be1.reshape(1, C), _stack_conv_w(w2), b2.reshape(C, 1),
                g2.reshape(1, C), be2.reshape(1, C), lw.reshape(1, C),
                lb.reshape(1, 1))

    dp = prep_vp(dp_w1, dp_b1, dp_g1, dp_be1, dp_w2, dp_b2, dp_g2, dp_be2, dp_lw, dp_lb)
    pp = prep_vp(pp_w1, pp_b1, pp_g1, pp_be1, pp_w2, pp_b2, pp_g2, pp_be2, pp_lw, pp_lb)
    ep = prep_vp(ep_w1, ep_b1, ep_g1, ep_be1, ep_w2, ep_b2, ep_g2, ep_be2, ep_lw, ep_lb)
    pew = pe_w[:, 0, :].T
    eew = ee_w[:, 0, :].T

    vp_specs = [_full((3 * C, C)), _full((C, 1)), _full((1, C)), _full((1, C)),
                _full((3 * C, C)), _full((C, 1)), _full((1, C)), _full((1, C)),
                _full((1, C)), _full((1, 1))]

    in_specs = ([pl.BlockSpec(memory_space=pltpu.SMEM),
                 pl.BlockSpec((1, C, T), lambda b: (b, 0, 0)),
                 pl.BlockSpec((1, 1, T), lambda b: (b, 0, 0)),
                 pl.BlockSpec((1, T, 1), lambda b: (b, 0, 0)),
                 pl.BlockSpec((1, T, 1), lambda b: (b, 0, 0))]
                + vp_specs * 3
                + [_full((3, C)), _full((1, C)), _full((3, C)), _full((1, C))])

    out_shapes = (
        jax.ShapeDtypeStruct((B, C, T), F32),
        jax.ShapeDtypeStruct((B, 1, C), jnp.int32),
        jax.ShapeDtypeStruct((B, 1, C), F32),
        jax.ShapeDtypeStruct((B, 1, C), F32),
        jax.ShapeDtypeStruct((B, 1, C), F32),
    )
    out_specs = (
        pl.BlockSpec((1, C, T), lambda b: (b, 0, 0)),
        pl.BlockSpec((1, 1, C), lambda b: (b, 0, 0)),
        pl.BlockSpec((1, 1, C), lambda b: (b, 0, 0)),
        pl.BlockSpec((1, 1, C), lambda b: (b, 0, 0)),
        pl.BlockSpec((1, 1, C), lambda b: (b, 0, 0)),
    )

    out, mel, dpred, ppred, epred = pl.pallas_call(
        _body,
        grid=(B,),
        in_specs=in_specs,
        out_specs=out_specs,
        out_shape=out_shapes,
        compiler_params=pltpu.CompilerParams(
            dimension_semantics=("arbitrary",)),
    )(maxlen, x, dur3, pt3, et3, *dp, *pp, *ep, pew, pe_b.reshape(1, C),
      eew, ee_b.reshape(1, C))

    return (out, mel[:, 0, 0], dpred.reshape(B, C), ppred.reshape(B, C),
            epred.reshape(B, C))


# speedup vs baseline: 2.0332x; 2.0332x over previous
import jax
import jax.numpy as jnp
from jax import lax
from jax.experimental import pallas as pl
from jax.experimental.pallas import tpu as pltpu

F32 = jnp.float32
B, C, T = 16, 256, 256
HIGH = lax.Precision.DEFAULT


def _conv_mm(h, w_ref, b_ref):
    hm1 = jnp.concatenate([jnp.zeros((C, 1), F32), h[:, :-1]], axis=1)
    hp1 = jnp.concatenate([h[:, 1:], jnp.zeros((C, 1), F32)], axis=1)
    acc = jnp.dot(w_ref[0:C, :], hm1, preferred_element_type=F32, precision=HIGH)
    acc = acc + jnp.dot(w_ref[C:2 * C, :], h, preferred_element_type=F32, precision=HIGH)
    acc = acc + jnp.dot(w_ref[2 * C:3 * C, :], hp1, preferred_element_type=F32, precision=HIGH)
    return acc + b_ref[:]


def _layer_norm(h, g_ref, be_ref):
    mu = jnp.mean(h, axis=1, keepdims=True)
    hc = h - mu
    var = jnp.mean(hc * hc, axis=1, keepdims=True)
    return hc * lax.rsqrt(var + 1e-5) * g_ref[:] + be_ref[:]


def _vp(h0, w1, b1, g1, be1, w2, b2, g2, be2, lw, lb):
    h = jnp.maximum(_conv_mm(h0, w1, b1), 0.0)
    h = _layer_norm(h, g1, be1)
    h = jnp.maximum(_conv_mm(h, w2, b2), 0.0)
    h = _layer_norm(h, g2, be2)
    pred = lax.dot_general(lw[:], h, (((1,), (1,)), ((), ())),
                           preferred_element_type=F32, precision=HIGH)
    return pred + lb[:]


def _emb(t_col, w_ref, b_ref):
    tm1 = jnp.concatenate([jnp.zeros((1, 1), F32), t_col[:-1, :]], axis=0)
    tp1 = jnp.concatenate([t_col[1:, :], jnp.zeros((1, 1), F32)], axis=0)
    return (tm1 * w_ref[0:1, :] + t_col * w_ref[1:2, :] + tp1 * w_ref[2:3, :]
            + b_ref[:])


def _body(maxlen_ref, x_ref, dur_ref, pt_ref, et_ref,
          dw1, db1, dg1, dbe1, dw2, db2, dg2, dbe2, dlw, dlb,
          pw1, pb1, pg1, pbe1, pw2, pb2, pg2, pbe2, plw, plb,
          ew1, eb1, eg1, ebe1, ew2, eb2, eg2, ebe2, elw, elb,
          pew, peb, eew, eeb,
          out_ref, mel_ref, dpred_ref, ppred_ref, epred_ref):
    xb = x_ref[0]

    dpred_ref[0] = _vp(xb, dw1, db1, dg1, dbe1, dw2, db2, dg2, dbe2, dlw, dlb)

    d = dur_ref[0].astype(F32)
    row_i = lax.broadcasted_iota(jnp.int32, (C, C), 0).astype(F32)
    col_i = lax.broadcasted_iota(jnp.int32, (C, C), 1).astype(F32)
    upper = (row_i <= col_i).astype(F32)
    cs = jnp.dot(d, upper, preferred_element_type=F32, precision=HIGH)
    total = jnp.sum(d)
    idx = jnp.sum((row_i >= cs).astype(F32), axis=1, keepdims=True)
    pcol = lax.broadcasted_iota(jnp.int32, (C, 1), 0).astype(F32)
    maxlen_f = maxlen_ref[0].astype(F32)
    valid = (pcol < total) & (pcol < maxlen_f)
    onehot = ((idx == col_i) & valid).astype(F32)
    x2b = jnp.dot(onehot, xb, preferred_element_type=F32, precision=HIGH)
    mel_ref[0] = jnp.full((1, C), total, F32).astype(jnp.int32)

    ppred_ref[0] = _vp(x2b, pw1, pb1, pg1, pbe1, pw2, pb2, pg2, pbe2, plw, plb)
    epred_ref[0] = _vp(x2b, ew1, eb1, eg1, ebe1, ew2, eb2, eg2, ebe2, elw, elb)

    pemb = _emb(pt_ref[0], pew, peb)
    eemb = _emb(et_ref[0], eew, eeb)
    out_ref[0] = x2b + pemb + eemb


def _stack_conv_w(w):
    return jnp.concatenate([w[:, :, 0], w[:, :, 1], w[:, :, 2]], axis=0)


def _full(shape):
    nd = len(shape)
    return pl.BlockSpec(shape, lambda b: (0,) * nd)


def kernel(x, src_len, duration_target, pitch_target, energy_target, max_len,
           dp_w1, dp_b1, dp_g1, dp_be1, dp_w2, dp_b2, dp_g2, dp_be2, dp_lw, dp_lb,
           pp_w1, pp_b1, pp_g1, pp_be1, pp_w2, pp_b2, pp_g2, pp_be2, pp_lw, pp_lb,
           ep_w1, ep_b1, ep_g1, ep_be1, ep_w2, ep_b2, ep_g2, ep_be2, ep_lw, ep_lb,
           pe_w, pe_b, ee_w, ee_b):
    del src_len
    dur3 = duration_target.astype(jnp.int32).reshape(B, 1, T)
    pt3 = pitch_target.astype(F32).reshape(B, T, 1)
    et3 = energy_target.astype(F32).reshape(B, T, 1)
    maxlen = jnp.asarray(max_len, jnp.int32).reshape(1)

    def prep_vp(w1, b1, g1, be1, w2, b2, g2, be2, lw, lb):
        return (_stack_conv_w(w1), b1.reshape(C, 1), g1.reshape(1, C),
                be1.reshape(1, C), _stack_conv_w(w2), b2.reshape(C, 1),
                g2.reshape(1, C), be2.reshape(1, C), lw.reshape(1, C),
                lb.reshape(1, 1))

    dp = prep_vp(dp_w1, dp_b1, dp_g1, dp_be1, dp_w2, dp_b2, dp_g2, dp_be2, dp_lw, dp_lb)
    pp = prep_vp(pp_w1, pp_b1, pp_g1, pp_be1, pp_w2, pp_b2, pp_g2, pp_be2, pp_lw, pp_lb)
    ep = prep_vp(ep_w1, ep_b1, ep_g1, ep_be1, ep_w2, ep_b2, ep_g2, ep_be2, ep_lw, ep_lb)
    pew = pe_w[:, 0, :].T
    eew = ee_w[:, 0, :].T

    vp_specs = [_full((3 * C, C)), _full((C, 1)), _full((1, C)), _full((1, C)),
                _full((3 * C, C)), _full((C, 1)), _full((1, C)), _full((1, C)),
                _full((1, C)), _full((1, 1))]

    in_specs = ([pl.BlockSpec(memory_space=pltpu.SMEM),
                 pl.BlockSpec((1, C, T), lambda b: (b, 0, 0)),
                 pl.BlockSpec((1, 1, T), lambda b: (b, 0, 0)),
                 pl.BlockSpec((1, T, 1), lambda b: (b, 0, 0)),
                 pl.BlockSpec((1, T, 1), lambda b: (b, 0, 0))]
                + vp_specs * 3
                + [_full((3, C)), _full((1, C)), _full((3, C)), _full((1, C))])

    out_shapes = (
        jax.ShapeDtypeStruct((B, C, T), F32),
        jax.ShapeDtypeStruct((B, 1, C), jnp.int32),
        jax.ShapeDtypeStruct((B, 1, C), F32),
        jax.ShapeDtypeStruct((B, 1, C), F32),
        jax.ShapeDtypeStruct((B, 1, C), F32),
    )
    out_specs = (
        pl.BlockSpec((1, C, T), lambda b: (b, 0, 0)),
        pl.BlockSpec((1, 1, C), lambda b: (b, 0, 0)),
        pl.BlockSpec((1, 1, C), lambda b: (b, 0, 0)),
        pl.BlockSpec((1, 1, C), lambda b: (b, 0, 0)),
        pl.BlockSpec((1, 1, C), lambda b: (b, 0, 0)),
    )

    out, mel, dpred, ppred, epred = pl.pallas_call(
        _body,
        grid=(B,),
        in_specs=in_specs,
        out_specs=out_specs,
        out_shape=out_shapes,
        compiler_params=pltpu.CompilerParams(
            dimension_semantics=("parallel",)),
    )(maxlen, x, dur3, pt3, et3, *dp, *pp, *ep, pew, pe_b.reshape(1, C),
      eew, ee_b.reshape(1, C))

    return (out, mel[:, 0, 0], dpred.reshape(B, C), ppred.reshape(B, C),
            epred.reshape(B, C))
